# baseline (speedup 1.0000x reference)
import functools

import jax
import jax.numpy as jnp
from jax import lax
from jax.experimental import pallas as pl
from jax.experimental.pallas import tpu as pltpu
from jax.experimental.pallas import tpu_sc as plsc

N_NODES = 10000
N_EDGES = 320000
D = 128
N_GRAPHS = 64

NC = 2
NS = 16
NW = NC * NS
CHUNK = 128
NCHUNKS = 80
EPW = NCHUNKS * CHUNK
E_PAD = NW * EPW
N_PAD = 10240
ROWS_PER_S = N_PAD // NS
DEG_W = 128

_vmesh = plsc.VectorSubcoreMesh(core_axis_name="c", subcore_axis_name="s")


@jax.jit
def _sc_degree(dstb, ones_rows, zero_rows):
    @functools.partial(
        pl.kernel,
        out_type=jax.ShapeDtypeStruct((NC, N_PAD, DEG_W), jnp.float32),
        mesh=_vmesh,
        scratch_types=[
            pltpu.VMEM((NCHUNKS, CHUNK), jnp.int32),
            pltpu.VMEM((CHUNK, DEG_W), jnp.float32),
            pltpu.VMEM_SHARED((N_PAD, DEG_W), jnp.float32),
        ],
    )
    def k(dst_hbm, ones_hbm, zeros_hbm, out_hbm, di, ones_v, acc):
        c = lax.axis_index("c")
        s = lax.axis_index("s")
        pltpu.sync_copy(zeros_hbm, acc.at[pl.ds(s * ROWS_PER_S, ROWS_PER_S)])
        pltpu.sync_copy(ones_hbm, ones_v)
        pltpu.sync_copy(dst_hbm.at[c * NS + s], di)
        plsc.subcore_barrier()

        @pl.loop(0, NCHUNKS)
        def _(j):
            pltpu.sync_copy(ones_v, acc.at[di.at[j]], add=True)

        plsc.subcore_barrier()
        sl = pl.ds(s * ROWS_PER_S, ROWS_PER_S)
        pltpu.sync_copy(acc.at[sl], out_hbm.at[c].at[sl])

    return k(dstb, ones_rows, zero_rows)


@jax.jit
def _sc_edge_agg(y, srcb, dstb, zero_rows):
    @functools.partial(
        pl.kernel,
        out_type=jax.ShapeDtypeStruct((NC, N_PAD, D), jnp.float32),
        mesh=_vmesh,
        scratch_types=[
            pltpu.VMEM((NCHUNKS, CHUNK), jnp.int32),
            pltpu.VMEM((NCHUNKS, CHUNK), jnp.int32),
            pltpu.VMEM((CHUNK, D), jnp.float32),
            pltpu.VMEM_SHARED((N_PAD, D), jnp.float32),
        ],
    )
    def k(y_hbm, src_hbm, dst_hbm, zeros_hbm, out_hbm, si, di, rows, acc):
        c = lax.axis_index("c")
        s = lax.axis_index("s")
        w = c * NS + s
        pltpu.sync_copy(zeros_hbm, acc.at[pl.ds(s * ROWS_PER_S, ROWS_PER_S)])
        pltpu.sync_copy(src_hbm.at[w], si)
        pltpu.sync_copy(dst_hbm.at[w], di)
        plsc.subcore_barrier()

        @pl.loop(0, NCHUNKS)
        def _(j):
            pltpu.sync_copy(y_hbm.at[si.at[j]], rows)
            pltpu.sync_copy(rows, acc.at[di.at[j]], add=True)

        plsc.subcore_barrier()
        sl = pl.ds(s * ROWS_PER_S, ROWS_PER_S)
        pltpu.sync_copy(acc.at[sl], out_hbm.at[c].at[sl])

    return k(y, srcb, dstb, zero_rows)


def _tc0_body(degp_ref, x_ref, w1_ref, y_ref, dinv_ref):
    deg = (degp_ref[0, :N_NODES, :1] + degp_ref[1, :N_NODES, :1]
           + 1.0)
    dinv = lax.rsqrt(deg)
    dinv_ref[...] = dinv
    y_ref[...] = jnp.dot(x_ref[...], w1_ref[...],
                         preferred_element_type=jnp.float32) * dinv


@jax.jit
def _tc0(deg_partials, x, W1):
    return pl.pallas_call(
        _tc0_body,
        out_shape=(
            jax.ShapeDtypeStruct((N_NODES, D), jnp.float32),
            jax.ShapeDtypeStruct((N_NODES, 1), jnp.float32),
        ),
    )(deg_partials, x, W1)


def _tc_mid_body(p_ref, y_ref, dinv_ref, b_ref, w_ref, o_ref):
    s = p_ref[0, :N_NODES] + p_ref[1, :N_NODES] + y_ref[...]
    f = jnp.maximum(s * dinv_ref[...] + b_ref[...], 0.0)
    o_ref[...] = jnp.dot(f, w_ref[...],
                         preferred_element_type=jnp.float32) * dinv_ref[...]


@jax.jit
def _tc_mid(partials, y, dinv, b_row, W_next):
    return pl.pallas_call(
        _tc_mid_body,
        out_shape=jax.ShapeDtypeStruct((N_NODES, D), jnp.float32),
    )(partials, y, dinv, b_row, W_next)


def _tc_final_body(p_ref, y_ref, dinv_ref, b_ref, batch_ref, fcw_ref, fcb_ref,
                   o_ref):
    s = p_ref[0, :N_NODES] + p_ref[1, :N_NODES] + y_ref[...]
    f = jnp.maximum(s * dinv_ref[...] + b_ref[...], 0.0)
    gids = lax.broadcasted_iota(jnp.int32, (N_NODES, N_GRAPHS), 1)
    onehot = (batch_ref[...] == gids).astype(jnp.float32)
    sums = lax.dot_general(onehot, f, (((0,), (0,)), ((), ())),
                           preferred_element_type=jnp.float32,
                           precision=lax.Precision.HIGHEST)
    cnts = lax.dot_general(onehot, jnp.ones((N_NODES, 1), jnp.float32),
                           (((0,), (0,)), ((), ())),
                           preferred_element_type=jnp.float32)
    g = sums / jnp.maximum(cnts, 1.0)
    o_ref[...] = jnp.dot(g, fcw_ref[...],
                         preferred_element_type=jnp.float32) + fcb_ref[...]


@jax.jit
def _tc_final(partials, y, dinv, b_row, batch_col, fcW, fcb_row):
    return pl.pallas_call(
        _tc_final_body,
        out_shape=jax.ShapeDtypeStruct((N_GRAPHS, 1), jnp.float32),
    )(partials, y, dinv, b_row, batch_col, fcW, fcb_row)


def kernel(x, edge_index, batch, W1, b1, W2, b2, W3, b3, fcW, fcb):
    pad = E_PAD - N_EDGES
    pad_dst = (N_NODES + jnp.arange(pad) % (N_PAD - N_NODES)).astype(jnp.int32)
    src = jnp.concatenate(
        [edge_index[0].astype(jnp.int32), jnp.zeros((pad,), jnp.int32)]
    ).reshape(NW, NCHUNKS, CHUNK)
    dst = jnp.concatenate(
        [edge_index[1].astype(jnp.int32), pad_dst]
    ).reshape(NW, NCHUNKS, CHUNK)
    batch_col = batch.astype(jnp.int32).reshape(N_NODES, 1)

    zeros_deg = jnp.zeros((ROWS_PER_S, DEG_W), jnp.float32)
    ones_deg = jnp.ones((CHUNK, DEG_W), jnp.float32)
    zeros_acc = jnp.zeros((ROWS_PER_S, D), jnp.float32)

    deg_partials = _sc_degree(dst, ones_deg, zeros_deg)
    y1, dinv = _tc0(deg_partials, x, W1)

    p1 = _sc_edge_agg(y1, src, dst, zeros_acc)
    y2 = _tc_mid(p1, y1, dinv, b1.reshape(1, D), W2)

    p2 = _sc_edge_agg(y2, src, dst, zeros_acc)
    y3 = _tc_mid(p2, y2, dinv, b2.reshape(1, D), W3)

    p3 = _sc_edge_agg(y3, src, dst, zeros_acc)
    return _tc_final(p3, y3, dinv, b3.reshape(1, D), batch_col, fcW,
                     fcb.reshape(1, 1))


# speedup vs baseline: 2.4295x; 2.4295x over previous
import functools

import jax
import jax.numpy as jnp
from jax import lax
from jax.experimental import pallas as pl
from jax.experimental.pallas import tpu as pltpu
from jax.experimental.pallas import tpu_sc as plsc

N_NODES = 10000
N_EDGES = 320000
D = 128
N_GRAPHS = 64

NC = 2
NS = 16
NW = NC * NS
EPW_E = N_EDGES // NW
ECHUNK = 200
CHUNK = 128
NCHUNKS = 80
EPW = NCHUNKS * CHUNK
E_PAD = NW * EPW
N_PAD = 10240
ROWS_PER_S = N_PAD // NS
DEG_W = 128

_vmesh = plsc.VectorSubcoreMesh(core_axis_name="c", subcore_axis_name="s")


@jax.jit
def _sc_degree(dstb, ones_rows, zero_rows):
    @functools.partial(
        pl.kernel,
        out_type=jax.ShapeDtypeStruct((NC, N_PAD, DEG_W), jnp.float32),
        mesh=_vmesh,
        scratch_types=[
            pltpu.VMEM((NCHUNKS, CHUNK), jnp.int32),
            pltpu.VMEM((CHUNK, DEG_W), jnp.float32),
            pltpu.VMEM_SHARED((N_PAD, DEG_W), jnp.float32),
        ],
    )
    def k(dst_hbm, ones_hbm, zeros_hbm, out_hbm, di, ones_v, acc):
        c = lax.axis_index("c")
        s = lax.axis_index("s")
        pltpu.sync_copy(zeros_hbm, acc.at[pl.ds(s * ROWS_PER_S, ROWS_PER_S)])
        pltpu.sync_copy(ones_hbm, ones_v)
        pltpu.sync_copy(dst_hbm.at[c * NS + s], di)
        plsc.subcore_barrier()

        @pl.loop(0, NCHUNKS)
        def _(j):
            pltpu.sync_copy(ones_v, acc.at[di.at[j]], add=True)

        plsc.subcore_barrier()
        sl = pl.ds(s * ROWS_PER_S, ROWS_PER_S)
        pltpu.sync_copy(acc.at[sl], out_hbm.at[c].at[sl])

    return k(dstb, ones_rows, zero_rows)


@jax.jit
def _sc_edge_agg(y, srcb, dstb, zero_rows):
    @functools.partial(
        pl.kernel,
        out_type=jax.ShapeDtypeStruct((NC, N_PAD, D), jnp.float32),
        mesh=_vmesh,
        scratch_types=[
            pltpu.VMEM((ECHUNK,), jnp.int32),
            pltpu.VMEM((ECHUNK,), jnp.int32),
            pltpu.VMEM((ECHUNK, D), jnp.float32),
            pltpu.VMEM_SHARED((N_PAD, D), jnp.float32),
        ],
    )
    def k(y_hbm, src_hbm, dst_hbm, zeros_hbm, out_hbm, si, di, rows, acc):
        c = lax.axis_index("c")
        s = lax.axis_index("s")
        pltpu.sync_copy(zeros_hbm, acc.at[pl.ds(s * ROWS_PER_S, ROWS_PER_S)])
        plsc.subcore_barrier()
        base = (c * NS + s) * EPW_E

        @pl.loop(0, EPW_E, step=ECHUNK)
        def _(off):
            pltpu.sync_copy(src_hbm.at[pl.ds(base + off, ECHUNK)], si)
            pltpu.sync_copy(dst_hbm.at[pl.ds(base + off, ECHUNK)], di)
            pltpu.sync_copy(y_hbm.at[si], rows)
            pltpu.sync_copy(rows, acc.at[di], add=True)

        plsc.subcore_barrier()
        sl = pl.ds(s * ROWS_PER_S, ROWS_PER_S)
        pltpu.sync_copy(acc.at[sl], out_hbm.at[c].at[sl])

    return k(y, srcb, dstb, zero_rows)


def _tc0_body(degp_ref, x_ref, w1_ref, y_ref, dinv_ref):
    deg = (degp_ref[0, :N_NODES, :1] + degp_ref[1, :N_NODES, :1]
           + 1.0)
    dinv = lax.rsqrt(deg)
    dinv_ref[...] = dinv
    y_ref[...] = jnp.dot(x_ref[...], w1_ref[...],
                         preferred_element_type=jnp.float32) * dinv


@jax.jit
def _tc0(deg_partials, x, W1):
    return pl.pallas_call(
        _tc0_body,
        out_shape=(
            jax.ShapeDtypeStruct((N_NODES, D), jnp.float32),
            jax.ShapeDtypeStruct((N_NODES, 1), jnp.float32),
        ),
    )(deg_partials, x, W1)


def _tc_mid_body(p_ref, y_ref, dinv_ref, b_ref, w_ref, o_ref):
    s = p_ref[0, :N_NODES] + p_ref[1, :N_NODES] + y_ref[...]
    f = jnp.maximum(s * dinv_ref[...] + b_ref[...], 0.0)
    o_ref[...] = jnp.dot(f, w_ref[...],
                         preferred_element_type=jnp.float32) * dinv_ref[...]


@jax.jit
def _tc_mid(partials, y, dinv, b_row, W_next):
    return pl.pallas_call(
        _tc_mid_body,
        out_shape=jax.ShapeDtypeStruct((N_NODES, D), jnp.float32),
    )(partials, y, dinv, b_row, W_next)


def _tc_final_body(p_ref, y_ref, dinv_ref, b_ref, batch_ref, fcw_ref, fcb_ref,
                   o_ref):
    s = p_ref[0, :N_NODES] + p_ref[1, :N_NODES] + y_ref[...]
    f = jnp.maximum(s * dinv_ref[...] + b_ref[...], 0.0)
    gids = lax.broadcasted_iota(jnp.int32, (N_NODES, N_GRAPHS), 1)
    onehot = (batch_ref[...] == gids).astype(jnp.float32)
    sums = lax.dot_general(onehot, f, (((0,), (0,)), ((), ())),
                           preferred_element_type=jnp.float32,
                           precision=lax.Precision.HIGHEST)
    cnts = lax.dot_general(onehot, jnp.ones((N_NODES, 1), jnp.float32),
                           (((0,), (0,)), ((), ())),
                           preferred_element_type=jnp.float32)
    g = sums / jnp.maximum(cnts, 1.0)
    o_ref[...] = jnp.dot(g, fcw_ref[...],
                         preferred_element_type=jnp.float32) + fcb_ref[...]


@jax.jit
def _tc_final(partials, y, dinv, b_row, batch_col, fcW, fcb_row):
    return pl.pallas_call(
        _tc_final_body,
        out_shape=jax.ShapeDtypeStruct((N_GRAPHS, 1), jnp.float32),
    )(partials, y, dinv, b_row, batch_col, fcW, fcb_row)


def kernel(x, edge_index, batch, W1, b1, W2, b2, W3, b3, fcW, fcb):
    src = edge_index[0].astype(jnp.int32)
    dst = edge_index[1].astype(jnp.int32)
    pad = E_PAD - N_EDGES
    pad_dst = (N_NODES + jnp.arange(pad) % (N_PAD - N_NODES)).astype(jnp.int32)
    dstb = jnp.concatenate([dst, pad_dst]).reshape(NW, NCHUNKS, CHUNK)
    batch_col = batch.astype(jnp.int32).reshape(N_NODES, 1)

    zeros_deg = jnp.zeros((ROWS_PER_S, DEG_W), jnp.float32)
    ones_deg = jnp.ones((CHUNK, DEG_W), jnp.float32)
    zeros_acc = jnp.zeros((ROWS_PER_S, D), jnp.float32)

    deg_partials = _sc_degree(dstb, ones_deg, zeros_deg)
    y1, dinv = _tc0(deg_partials, x, W1)

    p1 = _sc_edge_agg(y1, src, dst, zeros_acc)
    y2 = _tc_mid(p1, y1, dinv, b1.reshape(1, D), W2)

    p2 = _sc_edge_agg(y2, src, dst, zeros_acc)
    y3 = _tc_mid(p2, y2, dinv, b2.reshape(1, D), W3)

    p3 = _sc_edge_agg(y3, src, dst, zeros_acc)
    return _tc_final(p3, y3, dinv, b3.reshape(1, D), batch_col, fcW,
                     fcb.reshape(1, 1))
